# baseline (speedup 1.0000x reference)
import functools

import jax
import jax.numpy as jnp
from jax import lax
from jax.experimental import pallas as pl
from jax.experimental.pallas import tpu as pltpu
from jax.experimental.pallas import tpu_sc as plsc

N_NODES = 10000
NP = 10240
E = 320000
EP = 327680
D = 64
DH = D // 2
NUM_GRAPHS = 128
NC, NS = 2, 16
ROWS_PER_TILE = NP // NS
CHUNKS = EP // NS // 128
NBUF = 4


def _zero_fill(buf, n_rows, width):
    zero16 = jnp.zeros((16,), jnp.float32)
    cols = width // 16

    def body(i, _):
        r = i // cols
        c = (i % cols) * 16
        buf[r, pl.ds(c, 16)] = zero16
        return 0

    lax.fori_loop(0, n_rows * cols, body, 0)


def _scatter_kernel(g_hbm, src_hbm, dst_hbm, out_hbm, src_v, dst_v, rows_v,
                    zbuf_v, g_sh, acc_sh, sem):
    c = lax.axis_index("c")
    s = lax.axis_index("s")

    _zero_fill(zbuf_v, 64, DH)
    base = s * ROWS_PER_TILE
    pltpu.sync_copy(g_hbm.at[c, pl.ds(base, ROWS_PER_TILE)],
                    g_sh.at[pl.ds(base, ROWS_PER_TILE)])
    for k in range(ROWS_PER_TILE // 64):
        pltpu.sync_copy(zbuf_v, acc_sh.at[pl.ds(base + k * 64, 64)])

    pltpu.sync_copy(src_hbm.at[s], src_v)
    pltpu.sync_copy(dst_hbm.at[s], dst_v)
    plsc.subcore_barrier()

    for b in range(NBUF):
        pltpu.async_copy(g_sh.at[src_v.at[b]], rows_v.at[b], sem[b])

    def body(g, _):
        for b in range(NBUF):
            j = g * NBUF + b
            pltpu.make_async_copy(g_sh.at[src_v.at[j]], rows_v.at[b],
                                  sem[b]).wait()
            pltpu.sync_copy(rows_v.at[b], acc_sh.at[dst_v.at[j]], add=True)

            @pl.when(j + NBUF < CHUNKS)
            def _():
                pltpu.async_copy(g_sh.at[src_v.at[j + NBUF]], rows_v.at[b],
                                 sem[b])
        return 0

    lax.fori_loop(0, CHUNKS // NBUF, body, 0)
    plsc.subcore_barrier()

    pltpu.sync_copy(acc_sh.at[pl.ds(base, ROWS_PER_TILE)],
                    out_hbm.at[c, pl.ds(base, ROWS_PER_TILE)])


def _sc_scatter(g, src3, dst3):
    mesh = plsc.VectorSubcoreMesh(core_axis_name="c", subcore_axis_name="s")
    return pl.kernel(
        _scatter_kernel,
        mesh=mesh,
        compiler_params=pltpu.CompilerParams(use_tc_tiling_on_sc=False),
        out_type=jax.ShapeDtypeStruct((NC, NP, DH), jnp.float32),
        scratch_types=[
            pltpu.VMEM((CHUNKS, 128), jnp.int32),
            pltpu.VMEM((CHUNKS, 128), jnp.int32),
            pltpu.VMEM((NBUF, 128, DH), jnp.float32),
            pltpu.VMEM((64, DH), jnp.float32),
            pltpu.VMEM_SHARED((NP, DH), jnp.float32),
            pltpu.VMEM_SHARED((NP, DH), jnp.float32),
            [pltpu.SemaphoreType.DMA] * NBUF,
        ],
    )(g, src3, dst3)


def _deg_kernel(dst_hbm, out_hbm, dst_v, deg2_v, buf32_v, degbuf_v, idx_v,
                zbuf_v, acc_sh):
    c = lax.axis_index("c")
    s = lax.axis_index("s")
    zero16 = jnp.zeros((16,), jnp.float32)
    one16 = jnp.ones((16,), jnp.float32)
    iota16 = jnp.arange(16, dtype=jnp.int32)

    def z1(i, _):
        deg2_v[i, :] = zero16
        return 0

    lax.fori_loop(0, 640, z1, 0)

    def z2(i, _):
        zbuf_v[i, :] = zero16
        idx_v[i // 8, pl.ds((i % 8) * 16, 16)] = iota16 + i * 16
        return 0

    lax.fori_loop(0, 40, z2, 0)
    pltpu.sync_copy(zbuf_v, acc_sh.at[pl.ds(s * 40, 40)])
    pltpu.sync_copy(dst_hbm.at[s], dst_v)
    plsc.subcore_barrier()

    half16 = (CHUNKS // 2) * 8

    def hist(j, _):
        jj = c * half16 + j
        idx16 = dst_v[jj // 8, pl.ds((jj % 8) * 16, 16)]
        plsc.addupdate_scatter(
            deg2_v, [lax.shift_right_logical(idx16, 4),
                     jnp.bitwise_and(idx16, 15)], one16)
        return 0

    lax.fori_loop(0, half16, hist, 0)
    plsc.subcore_barrier()
    for k in range(5):
        pltpu.sync_copy(deg2_v.at[pl.ds(k * 128, 128)],
                        acc_sh.at[idx_v.at[k]], add=True)
    plsc.subcore_barrier()

    pltpu.sync_copy(acc_sh.at[pl.ds(s * 40, 40)], degbuf_v)

    def expand(r, _):
        v16 = degbuf_v[r, :]
        base = r * 512
        for j in range(32):
            plsc.store_scatter(buf32_v, [iota16 * 32 + (base + j)], v16)
        return 0

    lax.fori_loop(0, 40, expand, 0)
    pltpu.sync_copy(buf32_v, out_hbm.at[c, pl.ds(s * 20480, 20480)])


def _sc_deg(dst3):
    mesh = plsc.VectorSubcoreMesh(core_axis_name="c", subcore_axis_name="s")
    return pl.kernel(
        _deg_kernel,
        mesh=mesh,
        compiler_params=pltpu.CompilerParams(use_tc_tiling_on_sc=False,
                                             needs_layout_passes=False),
        out_type=jax.ShapeDtypeStruct((NC, NP * 32), jnp.float32),
        scratch_types=[
            pltpu.VMEM((CHUNKS, 128), jnp.int32),
            pltpu.VMEM((640, 16), jnp.float32),
            pltpu.VMEM((20480,), jnp.float32),
            pltpu.VMEM((40, 16), jnp.float32),
            pltpu.VMEM((5, 128), jnp.int32),
            pltpu.VMEM((40, 16), jnp.float32),
            pltpu.VMEM_SHARED((640, 16), jnp.float32),
        ],
    )(dst3)


NR = NP // 4


def _kron4(wsub, rows):
    t = jnp.concatenate([wsub] * 4, axis=0)
    t = jnp.concatenate([t] * 4, axis=1)
    ri = lax.broadcasted_iota(jnp.int32, t.shape, 0) // (rows // 4)
    ci = lax.broadcasted_iota(jnp.int32, t.shape, 1) // 32
    return jnp.where(ri == ci, t, 0.0)


def _dinv_packed(degp_ref):
    deg = degp_ref[0] + degp_ref[1] + 1.0
    return lax.rsqrt(deg)


def _g0_body(x_ref, w_ref, degp_ref, o_ref):
    dinv = _dinv_packed(degp_ref)
    x2 = x_ref[...]
    for h in range(2):
        bd = _kron4(w_ref[:, h * DH:(h + 1) * DH], 512)
        o_ref[h] = dinv * jnp.dot(x2, bd, preferred_element_type=jnp.float32)


def _edges_body(ei_ref, src_ref, dst_ref):
    src_ref[pl.ds(0, E)] = ei_ref[0, :]
    dst_ref[pl.ds(0, E)] = ei_ref[1, :]
    src_ref[pl.ds(E, EP - E)] = jnp.full((EP - E,), N_NODES, jnp.int32)
    dst_ref[pl.ds(E, EP - E)] = jnp.full((EP - E,), NP - 1, jnp.int32)


def _tc_edges(edge_index):
    return pl.pallas_call(
        _edges_body,
        out_shape=[
            jax.ShapeDtypeStruct((EP,), jnp.int32),
            jax.ShapeDtypeStruct((EP,), jnp.int32),
        ],
    )(edge_index)


def _tc_g0(x2, W0, degp):
    return pl.pallas_call(
        _g0_body,
        out_shape=jax.ShapeDtypeStruct((NC, NR, 128), jnp.float32),
    )(x2, W0, degp)


def _relu_halves(s_ref, g_ref, dinv, b_ref):
    rs = []
    for h in range(2):
        bh = jnp.concatenate([b_ref[:, h * DH:(h + 1) * DH]] * 4, axis=1)
        rs.append(jnp.maximum(dinv * (s_ref[h] + g_ref[h]) + bh, 0.0))
    return rs


def _mid_body(s_ref, g_ref, degp_ref, b_ref, w_ref, o_ref):
    dinv = _dinv_packed(degp_ref)
    r = _relu_halves(s_ref, g_ref, dinv, b_ref)
    for h in range(2):
        acc = jnp.zeros((NR, 128), jnp.float32)
        for i in range(2):
            bd = _kron4(w_ref[i * DH:(i + 1) * DH, h * DH:(h + 1) * DH], 128)
            acc += jnp.dot(r[i], bd, preferred_element_type=jnp.float32)
        o_ref[h] = dinv * acc


def _tc_mid(S, g, degp, b, W):
    return pl.pallas_call(
        _mid_body,
        out_shape=jax.ShapeDtypeStruct((NC, NR, 128), jnp.float32),
    )(S, g, degp, b, W)


def _final_body(s_ref, g_ref, degp_ref, b_ref, batchp_ref, wlin_ref, blin_ref,
                o_ref):
    dinv = _dinv_packed(degp_ref)
    r = _relu_halves(s_ref, g_ref, dinv, b_ref)
    gid = lax.broadcasted_iota(jnp.int32, (NUM_GRAPHS, NR), 0)
    sums = []
    cnts = jnp.zeros((NUM_GRAPHS, 1), jnp.float32)
    for k in range(4):
        oh = (gid == batchp_ref[k:k + 1, :]).astype(jnp.float32)
        sums.append([jnp.dot(oh, r[h][:, k * DH:(k + 1) * DH],
                             preferred_element_type=jnp.float32)
                     for h in range(2)])
        cnts += jnp.sum(oh, axis=1, keepdims=True)
    pooled = jnp.concatenate(
        [sums[0][0] + sums[1][0] + sums[2][0] + sums[3][0],
         sums[0][1] + sums[1][1] + sums[2][1] + sums[3][1]],
        axis=1) / jnp.maximum(cnts, 1.0)
    o_ref[...] = jnp.dot(pooled, wlin_ref[...],
                         preferred_element_type=jnp.float32) + blin_ref[...]


def _tc_final(S, g, degp, b, batchp, Wlin, blin):
    return pl.pallas_call(
        _final_body,
        out_shape=jax.ShapeDtypeStruct((NUM_GRAPHS, Wlin.shape[1]),
                                       jnp.float32),
    )(S, g, degp, b, batchp, Wlin, blin)


@jax.jit
def kernel(x, edge_index, batch, W0, b0, W1, b1, W2, b2, Wlin, blin):
    n = x.shape[0]
    xp = jnp.zeros((NP, x.shape[1]), x.dtype).at[:n].set(x)
    src, dst = _tc_edges(edge_index)
    src3 = src.reshape(NS, CHUNKS, 128)
    dst3 = dst.reshape(NS, CHUNKS, 128)
    bb = jnp.full((NP,), NUM_GRAPHS + 7, jnp.int32).at[:n].set(batch)
    batchp = bb.reshape(NR, 4).T

    def to_sc(a):
        return a.reshape(NC, NP, DH)

    def to_tc(a):
        return a.reshape(NC, NR, 128)

    degp = _sc_deg(dst3).reshape(NC, NR, 128)
    g0 = _tc_g0(xp.reshape(NR, 512), W0, degp)
    S0 = to_tc(_sc_scatter(to_sc(g0), src3, dst3))
    g1 = _tc_mid(S0, g0, degp, b0.reshape(1, D), W1)
    S1 = to_tc(_sc_scatter(to_sc(g1), src3, dst3))
    g2 = _tc_mid(S1, g1, degp, b1.reshape(1, D), W2)
    S2 = to_tc(_sc_scatter(to_sc(g2), src3, dst3))
    out = _tc_final(S2, g2, degp, b2.reshape(1, D), batchp,
                    Wlin, blin.reshape(1, -1))
    return out


# speedup vs baseline: 1.0010x; 1.0010x over previous
import functools

import jax
import jax.numpy as jnp
from jax import lax
from jax.experimental import pallas as pl
from jax.experimental.pallas import tpu as pltpu
from jax.experimental.pallas import tpu_sc as plsc

N_NODES = 10000
NP = 10240
E = 320000
EP = 327680
D = 64
DH = D // 2
NUM_GRAPHS = 128
NC, NS = 2, 16
ROWS_PER_TILE = NP // NS
CHUNKS = EP // NS // 128
NBUF = 8


def _zero_fill(buf, n_rows, width):
    zero16 = jnp.zeros((16,), jnp.float32)
    cols = width // 16

    def body(i, _):
        r = i // cols
        c = (i % cols) * 16
        buf[r, pl.ds(c, 16)] = zero16
        return 0

    lax.fori_loop(0, n_rows * cols, body, 0)


def _scatter_kernel(g_hbm, src_hbm, dst_hbm, out_hbm, src_v, dst_v, rows_v,
                    zbuf_v, g_sh, acc_sh, sem):
    c = lax.axis_index("c")
    s = lax.axis_index("s")

    _zero_fill(zbuf_v, 64, DH)
    base = s * ROWS_PER_TILE
    pltpu.sync_copy(g_hbm.at[c, pl.ds(base, ROWS_PER_TILE)],
                    g_sh.at[pl.ds(base, ROWS_PER_TILE)])
    for k in range(ROWS_PER_TILE // 64):
        pltpu.sync_copy(zbuf_v, acc_sh.at[pl.ds(base + k * 64, 64)])

    pltpu.sync_copy(src_hbm.at[s], src_v)
    pltpu.sync_copy(dst_hbm.at[s], dst_v)
    plsc.subcore_barrier()

    for b in range(NBUF):
        pltpu.async_copy(g_sh.at[src_v.at[b]], rows_v.at[b], sem[b])

    def body(g, _):
        for b in range(NBUF):
            j = g * NBUF + b
            pltpu.make_async_copy(g_sh.at[src_v.at[j]], rows_v.at[b],
                                  sem[b]).wait()
            pltpu.sync_copy(rows_v.at[b], acc_sh.at[dst_v.at[j]], add=True)

            @pl.when(j + NBUF < CHUNKS)
            def _():
                pltpu.async_copy(g_sh.at[src_v.at[j + NBUF]], rows_v.at[b],
                                 sem[b])
        return 0

    lax.fori_loop(0, CHUNKS // NBUF, body, 0)
    plsc.subcore_barrier()

    pltpu.sync_copy(acc_sh.at[pl.ds(base, ROWS_PER_TILE)],
                    out_hbm.at[c, pl.ds(base, ROWS_PER_TILE)])


def _sc_scatter(g, src3, dst3):
    mesh = plsc.VectorSubcoreMesh(core_axis_name="c", subcore_axis_name="s")
    return pl.kernel(
        _scatter_kernel,
        mesh=mesh,
        compiler_params=pltpu.CompilerParams(use_tc_tiling_on_sc=False),
        out_type=jax.ShapeDtypeStruct((NC, NP, DH), jnp.float32),
        scratch_types=[
            pltpu.VMEM((CHUNKS, 128), jnp.int32),
            pltpu.VMEM((CHUNKS, 128), jnp.int32),
            pltpu.VMEM((NBUF, 128, DH), jnp.float32),
            pltpu.VMEM((64, DH), jnp.float32),
            pltpu.VMEM_SHARED((NP, DH), jnp.float32),
            pltpu.VMEM_SHARED((NP, DH), jnp.float32),
            [pltpu.SemaphoreType.DMA] * NBUF,
        ],
    )(g, src3, dst3)


def _deg_kernel(dst_hbm, out_hbm, dst_v, deg2_v, buf32_v, degbuf_v, idx_v,
                zbuf_v, acc_sh):
    c = lax.axis_index("c")
    s = lax.axis_index("s")
    zero16 = jnp.zeros((16,), jnp.float32)
    one16 = jnp.ones((16,), jnp.float32)
    iota16 = jnp.arange(16, dtype=jnp.int32)

    def z1(i, _):
        deg2_v[i, :] = zero16
        return 0

    lax.fori_loop(0, 640, z1, 0)

    def z2(i, _):
        zbuf_v[i, :] = zero16
        idx_v[i // 8, pl.ds((i % 8) * 16, 16)] = iota16 + i * 16
        return 0

    lax.fori_loop(0, 40, z2, 0)
    pltpu.sync_copy(zbuf_v, acc_sh.at[pl.ds(s * 40, 40)])
    pltpu.sync_copy(dst_hbm.at[s], dst_v)
    plsc.subcore_barrier()

    half16 = (CHUNKS // 2) * 8

    def hist(j, _):
        jj = c * half16 + j
        idx16 = dst_v[jj // 8, pl.ds((jj % 8) * 16, 16)]
        plsc.addupdate_scatter(
            deg2_v, [lax.shift_right_logical(idx16, 4),
                     jnp.bitwise_and(idx16, 15)], one16)
        return 0

    lax.fori_loop(0, half16, hist, 0)
    plsc.subcore_barrier()
    for k in range(5):
        pltpu.sync_copy(deg2_v.at[pl.ds(k * 128, 128)],
                        acc_sh.at[idx_v.at[k]], add=True)
    plsc.subcore_barrier()

    pltpu.sync_copy(acc_sh.at[pl.ds(s * 40, 40)], degbuf_v)

    def expand(r, _):
        v16 = degbuf_v[r, :]
        base = r * 512
        for j in range(32):
            plsc.store_scatter(buf32_v, [iota16 * 32 + (base + j)], v16)
        return 0

    lax.fori_loop(0, 40, expand, 0)
    pltpu.sync_copy(buf32_v, out_hbm.at[c, pl.ds(s * 20480, 20480)])


def _sc_deg(dst3):
    mesh = plsc.VectorSubcoreMesh(core_axis_name="c", subcore_axis_name="s")
    return pl.kernel(
        _deg_kernel,
        mesh=mesh,
        compiler_params=pltpu.CompilerParams(use_tc_tiling_on_sc=False,
                                             needs_layout_passes=False),
        out_type=jax.ShapeDtypeStruct((NC, NP * 32), jnp.float32),
        scratch_types=[
            pltpu.VMEM((CHUNKS, 128), jnp.int32),
            pltpu.VMEM((640, 16), jnp.float32),
            pltpu.VMEM((20480,), jnp.float32),
            pltpu.VMEM((40, 16), jnp.float32),
            pltpu.VMEM((5, 128), jnp.int32),
            pltpu.VMEM((40, 16), jnp.float32),
            pltpu.VMEM_SHARED((640, 16), jnp.float32),
        ],
    )(dst3)


NR = NP // 4


def _kron4(wsub, rows):
    t = jnp.concatenate([wsub] * 4, axis=0)
    t = jnp.concatenate([t] * 4, axis=1)
    ri = lax.broadcasted_iota(jnp.int32, t.shape, 0) // (rows // 4)
    ci = lax.broadcasted_iota(jnp.int32, t.shape, 1) // 32
    return jnp.where(ri == ci, t, 0.0)


def _dinv_packed(degp_ref):
    deg = degp_ref[0] + degp_ref[1] + 1.0
    return lax.rsqrt(deg)


def _g0_body(x_ref, w_ref, degp_ref, o_ref):
    dinv = _dinv_packed(degp_ref)
    x2 = x_ref[...]
    for h in range(2):
        bd = _kron4(w_ref[:, h * DH:(h + 1) * DH], 512)
        o_ref[h] = dinv * jnp.dot(x2, bd, preferred_element_type=jnp.float32)


def _edges_body(ei_ref, src_ref, dst_ref):
    src_ref[pl.ds(0, E)] = ei_ref[0, :]
    dst_ref[pl.ds(0, E)] = ei_ref[1, :]
    src_ref[pl.ds(E, EP - E)] = jnp.full((EP - E,), N_NODES, jnp.int32)
    dst_ref[pl.ds(E, EP - E)] = jnp.full((EP - E,), NP - 1, jnp.int32)


def _tc_edges(edge_index):
    return pl.pallas_call(
        _edges_body,
        out_shape=[
            jax.ShapeDtypeStruct((EP,), jnp.int32),
            jax.ShapeDtypeStruct((EP,), jnp.int32),
        ],
    )(edge_index)


def _tc_g0(x2, W0, degp):
    return pl.pallas_call(
        _g0_body,
        out_shape=jax.ShapeDtypeStruct((NC, NR, 128), jnp.float32),
    )(x2, W0, degp)


def _relu_halves(s_ref, g_ref, dinv, b_ref):
    rs = []
    for h in range(2):
        bh = jnp.concatenate([b_ref[:, h * DH:(h + 1) * DH]] * 4, axis=1)
        rs.append(jnp.maximum(dinv * (s_ref[h] + g_ref[h]) + bh, 0.0))
    return rs


def _mid_body(s_ref, g_ref, degp_ref, b_ref, w_ref, o_ref):
    dinv = _dinv_packed(degp_ref)
    r = _relu_halves(s_ref, g_ref, dinv, b_ref)
    for h in range(2):
        acc = jnp.zeros((NR, 128), jnp.float32)
        for i in range(2):
            bd = _kron4(w_ref[i * DH:(i + 1) * DH, h * DH:(h + 1) * DH], 128)
            acc += jnp.dot(r[i], bd, preferred_element_type=jnp.float32)
        o_ref[h] = dinv * acc


def _tc_mid(S, g, degp, b, W):
    return pl.pallas_call(
        _mid_body,
        out_shape=jax.ShapeDtypeStruct((NC, NR, 128), jnp.float32),
    )(S, g, degp, b, W)


def _final_body(s_ref, g_ref, degp_ref, b_ref, batchp_ref, wlin_ref, blin_ref,
                o_ref):
    dinv = _dinv_packed(degp_ref)
    r = _relu_halves(s_ref, g_ref, dinv, b_ref)
    gid = lax.broadcasted_iota(jnp.int32, (NUM_GRAPHS, NR), 0)
    sums = []
    cnts = jnp.zeros((NUM_GRAPHS, 1), jnp.float32)
    for k in range(4):
        oh = (gid == batchp_ref[k:k + 1, :]).astype(jnp.float32)
        sums.append([jnp.dot(oh, r[h][:, k * DH:(k + 1) * DH],
                             preferred_element_type=jnp.float32)
                     for h in range(2)])
        cnts += jnp.sum(oh, axis=1, keepdims=True)
    pooled = jnp.concatenate(
        [sums[0][0] + sums[1][0] + sums[2][0] + sums[3][0],
         sums[0][1] + sums[1][1] + sums[2][1] + sums[3][1]],
        axis=1) / jnp.maximum(cnts, 1.0)
    o_ref[...] = jnp.dot(pooled, wlin_ref[...],
                         preferred_element_type=jnp.float32) + blin_ref[...]


def _tc_final(S, g, degp, b, batchp, Wlin, blin):
    return pl.pallas_call(
        _final_body,
        out_shape=jax.ShapeDtypeStruct((NUM_GRAPHS, Wlin.shape[1]),
                                       jnp.float32),
    )(S, g, degp, b, batchp, Wlin, blin)


@jax.jit
def kernel(x, edge_index, batch, W0, b0, W1, b1, W2, b2, Wlin, blin):
    n = x.shape[0]
    xp = jnp.zeros((NP, x.shape[1]), x.dtype).at[:n].set(x)
    src, dst = _tc_edges(edge_index)
    src3 = src.reshape(NS, CHUNKS, 128)
    dst3 = dst.reshape(NS, CHUNKS, 128)
    bb = jnp.full((NP,), NUM_GRAPHS + 7, jnp.int32).at[:n].set(batch)
    batchp = bb.reshape(NR, 4).T

    def to_sc(a):
        return a.reshape(NC, NP, DH)

    def to_tc(a):
        return a.reshape(NC, NR, 128)

    degp = _sc_deg(dst3).reshape(NC, NR, 128)
    g0 = _tc_g0(xp.reshape(NR, 512), W0, degp)
    S0 = to_tc(_sc_scatter(to_sc(g0), src3, dst3))
    g1 = _tc_mid(S0, g0, degp, b0.reshape(1, D), W1)
    S1 = to_tc(_sc_scatter(to_sc(g1), src3, dst3))
    g2 = _tc_mid(S1, g1, degp, b1.reshape(1, D), W2)
    S2 = to_tc(_sc_scatter(to_sc(g2), src3, dst3))
    out = _tc_final(S2, g2, degp, b2.reshape(1, D), batchp,
                    Wlin, blin.reshape(1, -1))
    return out


# speedup vs baseline: 1.0022x; 1.0012x over previous
import functools

import jax
import jax.numpy as jnp
from jax import lax
from jax.experimental import pallas as pl
from jax.experimental.pallas import tpu as pltpu
from jax.experimental.pallas import tpu_sc as plsc

N_NODES = 10000
NP = 10240
E = 320000
EP = 327680
D = 64
DH = D // 2
NUM_GRAPHS = 128
NC, NS = 2, 16
ROWS_PER_TILE = NP // NS
CHUNKS = EP // NS // 128
NBUF = 4


def _zero_fill(buf, n_rows, width):
    zero16 = jnp.zeros((16,), jnp.float32)
    cols = width // 16

    def body(i, _):
        r = i // cols
        c = (i % cols) * 16
        buf[r, pl.ds(c, 16)] = zero16
        return 0

    lax.fori_loop(0, n_rows * cols, body, 0)


def _scatter_kernel(g_hbm, src_hbm, dst_hbm, out_hbm, src_v, dst_v, rows_v,
                    zbuf_v, g_sh, acc_sh, sem):
    c = lax.axis_index("c")
    s = lax.axis_index("s")

    _zero_fill(zbuf_v, 64, DH)
    base = s * ROWS_PER_TILE
    pltpu.sync_copy(g_hbm.at[c, pl.ds(base, ROWS_PER_TILE)],
                    g_sh.at[pl.ds(base, ROWS_PER_TILE)])
    for k in range(ROWS_PER_TILE // 64):
        pltpu.sync_copy(zbuf_v, acc_sh.at[pl.ds(base + k * 64, 64)])

    pltpu.sync_copy(src_hbm.at[s], src_v)
    pltpu.sync_copy(dst_hbm.at[s], dst_v)
    plsc.subcore_barrier()

    for b in range(NBUF):
        pltpu.async_copy(g_sh.at[src_v.at[b]], rows_v.at[b], sem[b])

    def body(g, _):
        for b in range(NBUF):
            j = g * NBUF + b
            pltpu.make_async_copy(g_sh.at[src_v.at[j]], rows_v.at[b],
                                  sem[b]).wait()
            pltpu.sync_copy(rows_v.at[b], acc_sh.at[dst_v.at[j]], add=True)

            @pl.when(j + NBUF < CHUNKS)
            def _():
                pltpu.async_copy(g_sh.at[src_v.at[j + NBUF]], rows_v.at[b],
                                 sem[b])
        return 0

    lax.fori_loop(0, CHUNKS // NBUF, body, 0)
    plsc.subcore_barrier()

    pltpu.sync_copy(acc_sh.at[pl.ds(base, ROWS_PER_TILE)],
                    out_hbm.at[c, pl.ds(base, ROWS_PER_TILE)])


def _sc_scatter(g, src3, dst3):
    mesh = plsc.VectorSubcoreMesh(core_axis_name="c", subcore_axis_name="s")
    return pl.kernel(
        _scatter_kernel,
        mesh=mesh,
        compiler_params=pltpu.CompilerParams(use_tc_tiling_on_sc=False),
        out_type=jax.ShapeDtypeStruct((NC, NP, DH), jnp.float32),
        scratch_types=[
            pltpu.VMEM((CHUNKS, 128), jnp.int32),
            pltpu.VMEM((CHUNKS, 128), jnp.int32),
            pltpu.VMEM((NBUF, 128, DH), jnp.float32),
            pltpu.VMEM((64, DH), jnp.float32),
            pltpu.VMEM_SHARED((NP, DH), jnp.float32),
            pltpu.VMEM_SHARED((NP, DH), jnp.float32),
            [pltpu.SemaphoreType.DMA] * NBUF,
        ],
    )(g, src3, dst3)


def _deg_kernel(dst_hbm, out_hbm, dst_v, deg2_v, buf32_v, degbuf_v, idx_v,
                zbuf_v, acc_sh):
    c = lax.axis_index("c")
    s = lax.axis_index("s")
    zero16 = jnp.zeros((16,), jnp.float32)
    one16 = jnp.ones((16,), jnp.float32)
    iota16 = jnp.arange(16, dtype=jnp.int32)

    def z1(i, _):
        deg2_v[i, :] = zero16
        return 0

    lax.fori_loop(0, 640, z1, 0)

    def z2(i, _):
        zbuf_v[i, :] = zero16
        idx_v[i // 8, pl.ds((i % 8) * 16, 16)] = iota16 + i * 16
        return 0

    lax.fori_loop(0, 40, z2, 0)
    pltpu.sync_copy(zbuf_v, acc_sh.at[pl.ds(s * 40, 40)])
    pltpu.sync_copy(dst_hbm.at[s], dst_v)
    plsc.subcore_barrier()

    half16 = (CHUNKS // 2) * 8

    def hist(j, _):
        jj = c * half16 + j
        idx16 = dst_v[jj // 8, pl.ds((jj % 8) * 16, 16)]
        plsc.addupdate_scatter(
            deg2_v, [lax.shift_right_logical(idx16, 4),
                     jnp.bitwise_and(idx16, 15)], one16)
        return 0

    lax.fori_loop(0, half16, hist, 0)
    plsc.subcore_barrier()
    for k in range(5):
        pltpu.sync_copy(deg2_v.at[pl.ds(k * 128, 128)],
                        acc_sh.at[idx_v.at[k]], add=True)
    plsc.subcore_barrier()

    pltpu.sync_copy(acc_sh.at[pl.ds(s * 40, 40)], degbuf_v)

    def expand(r, _):
        v16 = degbuf_v[r, :]
        base = r * 512
        for j in range(32):
            plsc.store_scatter(buf32_v, [iota16 * 32 + (base + j)], v16)
        return 0

    lax.fori_loop(0, 40, expand, 0)
    pltpu.sync_copy(buf32_v, out_hbm.at[c, pl.ds(s * 20480, 20480)])


def _sc_deg(dst3):
    mesh = plsc.VectorSubcoreMesh(core_axis_name="c", subcore_axis_name="s")
    return pl.kernel(
        _deg_kernel,
        mesh=mesh,
        compiler_params=pltpu.CompilerParams(use_tc_tiling_on_sc=False,
                                             needs_layout_passes=False),
        out_type=jax.ShapeDtypeStruct((NC, NP * 32), jnp.float32),
        scratch_types=[
            pltpu.VMEM((CHUNKS, 128), jnp.int32),
            pltpu.VMEM((640, 16), jnp.float32),
            pltpu.VMEM((20480,), jnp.float32),
            pltpu.VMEM((40, 16), jnp.float32),
            pltpu.VMEM((5, 128), jnp.int32),
            pltpu.VMEM((40, 16), jnp.float32),
            pltpu.VMEM_SHARED((640, 16), jnp.float32),
        ],
    )(dst3)


NR = NP // 4


def _kron4(wsub, rows):
    t = jnp.concatenate([wsub] * 4, axis=0)
    t = jnp.concatenate([t] * 4, axis=1)
    ri = lax.broadcasted_iota(jnp.int32, t.shape, 0) // (rows // 4)
    ci = lax.broadcasted_iota(jnp.int32, t.shape, 1) // 32
    return jnp.where(ri == ci, t, 0.0)


def _dinv_packed(degp_ref):
    deg = degp_ref[0] + degp_ref[1] + 1.0
    return lax.rsqrt(deg)


def _g0_body(x_ref, w_ref, degp_ref, o_ref):
    dinv = _dinv_packed(degp_ref)
    x2 = x_ref[...]
    for h in range(2):
        bd = _kron4(w_ref[:, h * DH:(h + 1) * DH], 512)
        o_ref[h] = dinv * jnp.dot(x2, bd, preferred_element_type=jnp.float32)


def _edges_body(ei_ref, src_ref, dst_ref):
    src_ref[pl.ds(0, E)] = ei_ref[0, :]
    dst_ref[pl.ds(0, E)] = ei_ref[1, :]
    src_ref[pl.ds(E, EP - E)] = jnp.full((EP - E,), N_NODES, jnp.int32)
    dst_ref[pl.ds(E, EP - E)] = jnp.full((EP - E,), NP - 1, jnp.int32)


def _tc_edges(edge_index):
    return pl.pallas_call(
        _edges_body,
        out_shape=[
            jax.ShapeDtypeStruct((EP,), jnp.int32),
            jax.ShapeDtypeStruct((EP,), jnp.int32),
        ],
    )(edge_index)


def _tc_g0(x2, W0, degp):
    return pl.pallas_call(
        _g0_body,
        out_shape=jax.ShapeDtypeStruct((NC, NR, 128), jnp.float32),
    )(x2, W0, degp)


def _relu_halves(s_ref, g_ref, dinv, b_ref):
    rs = []
    for h in range(2):
        bh = jnp.concatenate([b_ref[:, h * DH:(h + 1) * DH]] * 4, axis=1)
        rs.append(jnp.maximum(dinv * (s_ref[h] + g_ref[h]) + bh, 0.0))
    return rs


def _mid_body(s_ref, g_ref, degp_ref, b_ref, w_ref, o_ref):
    dinv = _dinv_packed(degp_ref)
    r = _relu_halves(s_ref, g_ref, dinv, b_ref)
    for h in range(2):
        acc = jnp.zeros((NR, 128), jnp.float32)
        for i in range(2):
            bd = _kron4(w_ref[i * DH:(i + 1) * DH, h * DH:(h + 1) * DH], 128)
            acc += jnp.dot(r[i], bd, preferred_element_type=jnp.float32)
        o_ref[h] = dinv * acc


def _tc_mid(S, g, degp, b, W):
    return pl.pallas_call(
        _mid_body,
        out_shape=jax.ShapeDtypeStruct((NC, NR, 128), jnp.float32),
    )(S, g, degp, b, W)


def _final_body(s_ref, g_ref, degp_ref, b_ref, batchp_ref, wlin_ref, blin_ref,
                o_ref):
    dinv = _dinv_packed(degp_ref)
    r = _relu_halves(s_ref, g_ref, dinv, b_ref)
    gid = lax.broadcasted_iota(jnp.int32, (NUM_GRAPHS, NR), 0)
    sums = []
    cnts = jnp.zeros((NUM_GRAPHS, 1), jnp.float32)
    for k in range(4):
        oh = (gid == batchp_ref[k:k + 1, :]).astype(jnp.float32)
        sums.append([jnp.dot(oh, r[h][:, k * DH:(k + 1) * DH],
                             preferred_element_type=jnp.float32)
                     for h in range(2)])
        cnts += jnp.sum(oh, axis=1, keepdims=True)
    pooled = jnp.concatenate(
        [sums[0][0] + sums[1][0] + sums[2][0] + sums[3][0],
         sums[0][1] + sums[1][1] + sums[2][1] + sums[3][1]],
        axis=1) / jnp.maximum(cnts, 1.0)
    o_ref[...] = jnp.dot(pooled, wlin_ref[...],
                         preferred_element_type=jnp.float32) + blin_ref[...]


def _tc_final(S, g, degp, b, batchp, Wlin, blin):
    return pl.pallas_call(
        _final_body,
        out_shape=jax.ShapeDtypeStruct((NUM_GRAPHS, Wlin.shape[1]),
                                       jnp.float32),
    )(S, g, degp, b, batchp, Wlin, blin)


@jax.jit
def kernel(x, edge_index, batch, W0, b0, W1, b1, W2, b2, Wlin, blin):
    n = x.shape[0]
    xp = jnp.zeros((NP, x.shape[1]), x.dtype).at[:n].set(x)
    src, dst = _tc_edges(edge_index)
    src3 = src.reshape(NS, CHUNKS, 128)
    dst3 = dst.reshape(NS, CHUNKS, 128)
    bb = jnp.full((NP,), NUM_GRAPHS + 7, jnp.int32).at[:n].set(batch)
    batchp = bb.reshape(NR, 4).T

    def to_sc(a):
        return a.reshape(NC, NP, DH)

    def to_tc(a):
        return a.reshape(NC, NR, 128)

    degp = _sc_deg(dst3).reshape(NC, NR, 128)
    g0 = _tc_g0(xp.reshape(NR, 512), W0, degp)
    S0 = to_tc(_sc_scatter(to_sc(g0), src3, dst3))
    g1 = _tc_mid(S0, g0, degp, b0.reshape(1, D), W1)
    S1 = to_tc(_sc_scatter(to_sc(g1), src3, dst3))
    g2 = _tc_mid(S1, g1, degp, b1.reshape(1, D), W2)
    S2 = to_tc(_sc_scatter(to_sc(g2), src3, dst3))
    out = _tc_final(S2, g2, degp, b2.reshape(1, D), batchp,
                    Wlin, blin.reshape(1, -1))
    return out


# speedup vs baseline: 1.0100x; 1.0078x over previous
import functools

import jax
import jax.numpy as jnp
from jax import lax
from jax.experimental import pallas as pl
from jax.experimental.pallas import tpu as pltpu
from jax.experimental.pallas import tpu_sc as plsc

N_NODES = 10000
NP = 10240
E = 320000
EP = 327680
D = 64
DH = D // 2
NUM_GRAPHS = 128
NC, NS = 2, 16
ROWS_PER_TILE = NP // NS
CHUNKS = EP // NS // 128
NBUF = 4


def _zero_fill(buf, n_rows, width):
    zero16 = jnp.zeros((16,), jnp.float32)
    cols = width // 16

    def body(i, _):
        r = i // cols
        c = (i % cols) * 16
        buf[r, pl.ds(c, 16)] = zero16
        return 0

    lax.fori_loop(0, n_rows * cols, body, 0)


def _scatter_kernel(g_hbm, src_hbm, dst_hbm, out_hbm, src_v, dst_v, rows_v,
                    zbuf_v, g_sh, acc_sh, sem):
    c = lax.axis_index("c")
    s = lax.axis_index("s")

    _zero_fill(zbuf_v, 64, DH)
    base = s * ROWS_PER_TILE
    pltpu.sync_copy(g_hbm.at[c, pl.ds(base, ROWS_PER_TILE)],
                    g_sh.at[pl.ds(base, ROWS_PER_TILE)])
    for k in range(ROWS_PER_TILE // 64):
        pltpu.sync_copy(zbuf_v, acc_sh.at[pl.ds(base + k * 64, 64)])

    pltpu.sync_copy(src_hbm.at[s], src_v)
    pltpu.sync_copy(dst_hbm.at[s], dst_v)
    plsc.subcore_barrier()

    for b in range(NBUF):
        pltpu.async_copy(g_sh.at[src_v.at[b]], rows_v.at[b], sem[b])

    def body(g, _):
        for b in range(NBUF):
            j = g * NBUF + b
            pltpu.make_async_copy(g_sh.at[src_v.at[j]], rows_v.at[b],
                                  sem[b]).wait()
            pltpu.sync_copy(rows_v.at[b], acc_sh.at[dst_v.at[j]], add=True)

            @pl.when(j + NBUF < CHUNKS)
            def _():
                pltpu.async_copy(g_sh.at[src_v.at[j + NBUF]], rows_v.at[b],
                                 sem[b])
        return 0

    lax.fori_loop(0, CHUNKS // NBUF, body, 0)
    plsc.subcore_barrier()

    pltpu.sync_copy(acc_sh.at[pl.ds(base, ROWS_PER_TILE)],
                    out_hbm.at[c, pl.ds(base, ROWS_PER_TILE)])


def _sc_scatter(g, src3, dst3):
    mesh = plsc.VectorSubcoreMesh(core_axis_name="c", subcore_axis_name="s")
    return pl.kernel(
        _scatter_kernel,
        mesh=mesh,
        compiler_params=pltpu.CompilerParams(use_tc_tiling_on_sc=False),
        out_type=jax.ShapeDtypeStruct((NC, NP, DH), jnp.float32),
        scratch_types=[
            pltpu.VMEM((CHUNKS, 128), jnp.int32),
            pltpu.VMEM((CHUNKS, 128), jnp.int32),
            pltpu.VMEM((NBUF, 128, DH), jnp.float32),
            pltpu.VMEM((64, DH), jnp.float32),
            pltpu.VMEM_SHARED((NP, DH), jnp.float32),
            pltpu.VMEM_SHARED((NP, DH), jnp.float32),
            [pltpu.SemaphoreType.DMA] * NBUF,
        ],
    )(g, src3, dst3)


def _deg_kernel(dst_hbm, out_hbm, dst_v, deg2_v, buf32_v, degbuf_v, idx_v,
                zbuf_v, acc_sh):
    c = lax.axis_index("c")
    s = lax.axis_index("s")
    zero16 = jnp.zeros((16,), jnp.float32)
    one16 = jnp.ones((16,), jnp.float32)
    iota16 = jnp.arange(16, dtype=jnp.int32)

    def z1(i, _):
        for m in range(8):
            deg2_v[i * 8 + m, :] = zero16
        return 0

    lax.fori_loop(0, 80, z1, 0)

    def z2(i, _):
        zbuf_v[i, :] = zero16
        idx_v[i // 8, pl.ds((i % 8) * 16, 16)] = iota16 + i * 16
        return 0

    lax.fori_loop(0, 40, z2, 0)
    pltpu.sync_copy(zbuf_v, acc_sh.at[pl.ds(s * 40, 40)])
    pltpu.sync_copy(dst_hbm.at[s], dst_v)
    plsc.subcore_barrier()

    half = CHUNKS // 2

    def hist(j, _):
        row = c * half + j
        for m in range(8):
            idx16 = dst_v[row, pl.ds(m * 16, 16)]
            plsc.addupdate_scatter(
                deg2_v, [lax.shift_right_logical(idx16, 4),
                         jnp.bitwise_and(idx16, 15)], one16)
        return 0

    lax.fori_loop(0, half, hist, 0)
    plsc.subcore_barrier()
    for k in range(5):
        pltpu.sync_copy(deg2_v.at[pl.ds(k * 128, 128)],
                        acc_sh.at[idx_v.at[k]], add=True)
    plsc.subcore_barrier()

    pltpu.sync_copy(acc_sh.at[pl.ds(s * 40, 40)], degbuf_v)

    def expand(r, _):
        v16 = degbuf_v[r, :]
        base = r * 512
        for j in range(32):
            plsc.store_scatter(buf32_v, [iota16 * 32 + (base + j)], v16)
        return 0

    lax.fori_loop(0, 40, expand, 0)
    pltpu.sync_copy(buf32_v, out_hbm.at[c, pl.ds(s * 20480, 20480)])


def _sc_deg(dst3):
    mesh = plsc.VectorSubcoreMesh(core_axis_name="c", subcore_axis_name="s")
    return pl.kernel(
        _deg_kernel,
        mesh=mesh,
        compiler_params=pltpu.CompilerParams(use_tc_tiling_on_sc=False,
                                             needs_layout_passes=False),
        out_type=jax.ShapeDtypeStruct((NC, NP * 32), jnp.float32),
        scratch_types=[
            pltpu.VMEM((CHUNKS, 128), jnp.int32),
            pltpu.VMEM((640, 16), jnp.float32),
            pltpu.VMEM((20480,), jnp.float32),
            pltpu.VMEM((40, 16), jnp.float32),
            pltpu.VMEM((5, 128), jnp.int32),
            pltpu.VMEM((40, 16), jnp.float32),
            pltpu.VMEM_SHARED((640, 16), jnp.float32),
        ],
    )(dst3)


NR = NP // 4


def _kron4(wsub, rows):
    t = jnp.concatenate([wsub] * 4, axis=0)
    t = jnp.concatenate([t] * 4, axis=1)
    ri = lax.broadcasted_iota(jnp.int32, t.shape, 0) // (rows // 4)
    ci = lax.broadcasted_iota(jnp.int32, t.shape, 1) // 32
    return jnp.where(ri == ci, t, 0.0)


def _dinv_packed(degp_ref):
    deg = degp_ref[0] + degp_ref[1] + 1.0
    return lax.rsqrt(deg)


def _g0_body(x_ref, w_ref, degp_ref, o_ref):
    dinv = _dinv_packed(degp_ref)
    x2 = x_ref[...]
    for h in range(2):
        bd = _kron4(w_ref[:, h * DH:(h + 1) * DH], 512)
        o_ref[h] = dinv * jnp.dot(x2, bd, preferred_element_type=jnp.float32)


def _edges_body(ei_ref, src_ref, dst_ref):
    src_ref[pl.ds(0, E)] = ei_ref[0, :]
    dst_ref[pl.ds(0, E)] = ei_ref[1, :]
    src_ref[pl.ds(E, EP - E)] = jnp.full((EP - E,), N_NODES, jnp.int32)
    dst_ref[pl.ds(E, EP - E)] = jnp.full((EP - E,), NP - 1, jnp.int32)


def _tc_edges(edge_index):
    return pl.pallas_call(
        _edges_body,
        out_shape=[
            jax.ShapeDtypeStruct((EP,), jnp.int32),
            jax.ShapeDtypeStruct((EP,), jnp.int32),
        ],
    )(edge_index)


def _tc_g0(x2, W0, degp):
    return pl.pallas_call(
        _g0_body,
        out_shape=jax.ShapeDtypeStruct((NC, NR, 128), jnp.float32),
    )(x2, W0, degp)


def _relu_halves(s_ref, g_ref, dinv, b_ref):
    rs = []
    for h in range(2):
        bh = jnp.concatenate([b_ref[:, h * DH:(h + 1) * DH]] * 4, axis=1)
        rs.append(jnp.maximum(dinv * (s_ref[h] + g_ref[h]) + bh, 0.0))
    return rs


def _mid_body(s_ref, g_ref, degp_ref, b_ref, w_ref, o_ref):
    dinv = _dinv_packed(degp_ref)
    r = _relu_halves(s_ref, g_ref, dinv, b_ref)
    for h in range(2):
        acc = jnp.zeros((NR, 128), jnp.float32)
        for i in range(2):
            bd = _kron4(w_ref[i * DH:(i + 1) * DH, h * DH:(h + 1) * DH], 128)
            acc += jnp.dot(r[i], bd, preferred_element_type=jnp.float32)
        o_ref[h] = dinv * acc


def _tc_mid(S, g, degp, b, W):
    return pl.pallas_call(
        _mid_body,
        out_shape=jax.ShapeDtypeStruct((NC, NR, 128), jnp.float32),
    )(S, g, degp, b, W)


def _final_body(s_ref, g_ref, degp_ref, b_ref, batchp_ref, wlin_ref, blin_ref,
                o_ref):
    dinv = _dinv_packed(degp_ref)
    r = _relu_halves(s_ref, g_ref, dinv, b_ref)
    gid = lax.broadcasted_iota(jnp.int32, (NUM_GRAPHS, NR), 0)
    sums = []
    cnts = jnp.zeros((NUM_GRAPHS, 1), jnp.float32)
    for k in range(4):
        oh = (gid == batchp_ref[k:k + 1, :]).astype(jnp.float32)
        sums.append([jnp.dot(oh, r[h][:, k * DH:(k + 1) * DH],
                             preferred_element_type=jnp.float32)
                     for h in range(2)])
        cnts += jnp.sum(oh, axis=1, keepdims=True)
    pooled = jnp.concatenate(
        [sums[0][0] + sums[1][0] + sums[2][0] + sums[3][0],
         sums[0][1] + sums[1][1] + sums[2][1] + sums[3][1]],
        axis=1) / jnp.maximum(cnts, 1.0)
    o_ref[...] = jnp.dot(pooled, wlin_ref[...],
                         preferred_element_type=jnp.float32) + blin_ref[...]


def _tc_final(S, g, degp, b, batchp, Wlin, blin):
    return pl.pallas_call(
        _final_body,
        out_shape=jax.ShapeDtypeStruct((NUM_GRAPHS, Wlin.shape[1]),
                                       jnp.float32),
    )(S, g, degp, b, batchp, Wlin, blin)


@jax.jit
def kernel(x, edge_index, batch, W0, b0, W1, b1, W2, b2, Wlin, blin):
    n = x.shape[0]
    xp = jnp.zeros((NP, x.shape[1]), x.dtype).at[:n].set(x)
    src, dst = _tc_edges(edge_index)
    src3 = src.reshape(NS, CHUNKS, 128)
    dst3 = dst.reshape(NS, CHUNKS, 128)
    bb = jnp.full((NP,), NUM_GRAPHS + 7, jnp.int32).at[:n].set(batch)
    batchp = bb.reshape(NR, 4).T

    def to_sc(a):
        return a.reshape(NC, NP, DH)

    def to_tc(a):
        return a.reshape(NC, NR, 128)

    degp = _sc_deg(dst3).reshape(NC, NR, 128)
    g0 = _tc_g0(xp.reshape(NR, 512), W0, degp)
    S0 = to_tc(_sc_scatter(to_sc(g0), src3, dst3))
    g1 = _tc_mid(S0, g0, degp, b0.reshape(1, D), W1)
    S1 = to_tc(_sc_scatter(to_sc(g1), src3, dst3))
    g2 = _tc_mid(S1, g1, degp, b1.reshape(1, D), W2)
    S2 = to_tc(_sc_scatter(to_sc(g2), src3, dst3))
    out = _tc_final(S2, g2, degp, b2.reshape(1, D), batchp,
                    Wlin, blin.reshape(1, -1))
    return out


# speedup vs baseline: 1.0101x; 1.0001x over previous
import jax
import jax.numpy as jnp
from jax import lax
from jax.experimental import pallas as pl
from jax.experimental.pallas import tpu as pltpu
from jax.experimental.pallas import tpu_sc as plsc

N_NODES = 10000
NP = 10240
E = 320000
EP = 327680
D = 64
DH = D // 2
NUM_GRAPHS = 128
NC, NS = 2, 16
ROWS_PER_TILE = NP // NS
CHUNKS = EP // NS // 128
NBUF = 4


def _zero_fill(buf, n_rows, width):
    zero16 = jnp.zeros((16,), jnp.float32)
    cols = width // 16

    def body(i, _):
        r = i // cols
        c = (i % cols) * 16
        buf[r, pl.ds(c, 16)] = zero16
        return 0

    lax.fori_loop(0, n_rows * cols, body, 0)


def _scatter_kernel(g_hbm, src_hbm, dst_hbm, out_hbm, src_v, dst_v, rows_v,
                    zbuf_v, g_sh, acc_sh, sem):
    c = lax.axis_index("c")
    s = lax.axis_index("s")

    _zero_fill(zbuf_v, 64, DH)
    base = s * ROWS_PER_TILE
    pltpu.sync_copy(g_hbm.at[c, pl.ds(base, ROWS_PER_TILE)],
                    g_sh.at[pl.ds(base, ROWS_PER_TILE)])
    for k in range(ROWS_PER_TILE // 64):
        pltpu.sync_copy(zbuf_v, acc_sh.at[pl.ds(base + k * 64, 64)])

    pltpu.sync_copy(src_hbm.at[s], src_v)
    pltpu.sync_copy(dst_hbm.at[s], dst_v)
    plsc.subcore_barrier()

    for b in range(NBUF):
        pltpu.async_copy(g_sh.at[src_v.at[b]], rows_v.at[b], sem[b])

    def body(g, _):
        for b in range(NBUF):
            j = g * NBUF + b
            pltpu.make_async_copy(g_sh.at[src_v.at[j]], rows_v.at[b],
                                  sem[b]).wait()
            pltpu.sync_copy(rows_v.at[b], acc_sh.at[dst_v.at[j]], add=True)

            @pl.when(j + NBUF < CHUNKS)
            def _():
                pltpu.async_copy(g_sh.at[src_v.at[j + NBUF]], rows_v.at[b],
                                 sem[b])
        return 0

    lax.fori_loop(0, CHUNKS // NBUF, body, 0)
    plsc.subcore_barrier()

    pltpu.sync_copy(acc_sh.at[pl.ds(base, ROWS_PER_TILE)],
                    out_hbm.at[c, pl.ds(base, ROWS_PER_TILE)])


def _sc_scatter(g, src3, dst3):
    mesh = plsc.VectorSubcoreMesh(core_axis_name="c", subcore_axis_name="s")
    return pl.kernel(
        _scatter_kernel,
        mesh=mesh,
        compiler_params=pltpu.CompilerParams(use_tc_tiling_on_sc=False),
        out_type=jax.ShapeDtypeStruct((NC, NP, DH), jnp.float32),
        scratch_types=[
            pltpu.VMEM((CHUNKS, 128), jnp.int32),
            pltpu.VMEM((CHUNKS, 128), jnp.int32),
            pltpu.VMEM((NBUF, 128, DH), jnp.float32),
            pltpu.VMEM((64, DH), jnp.float32),
            pltpu.VMEM_SHARED((NP, DH), jnp.float32),
            pltpu.VMEM_SHARED((NP, DH), jnp.float32),
            [pltpu.SemaphoreType.DMA] * NBUF,
        ],
    )(g, src3, dst3)


def _deg_kernel(dst_hbm, out_hbm, dst_v, deg2_v, buf32_v, degbuf_v, idx_v,
                zbuf_v, acc_sh):
    c = lax.axis_index("c")
    s = lax.axis_index("s")
    zero16 = jnp.zeros((16,), jnp.float32)
    one16 = jnp.ones((16,), jnp.float32)
    iota16 = jnp.arange(16, dtype=jnp.int32)

    def z1(i, _):
        for m in range(8):
            deg2_v[i * 8 + m, :] = zero16
        return 0

    lax.fori_loop(0, 80, z1, 0)

    def z2(i, _):
        zbuf_v[i, :] = zero16
        idx_v[i // 8, pl.ds((i % 8) * 16, 16)] = iota16 + i * 16
        return 0

    lax.fori_loop(0, 40, z2, 0)
    pltpu.sync_copy(zbuf_v, acc_sh.at[pl.ds(s * 40, 40)])
    pltpu.sync_copy(dst_hbm.at[s], dst_v)
    plsc.subcore_barrier()

    half = CHUNKS // 2

    def hist(j, _):
        row = c * half + j
        for m in range(8):
            idx16 = dst_v[row, pl.ds(m * 16, 16)]
            plsc.addupdate_scatter(
                deg2_v, [lax.shift_right_logical(idx16, 4),
                         jnp.bitwise_and(idx16, 15)], one16)
        return 0

    lax.fori_loop(0, half, hist, 0)
    plsc.subcore_barrier()
    for k in range(5):
        pltpu.sync_copy(deg2_v.at[pl.ds(k * 128, 128)],
                        acc_sh.at[idx_v.at[k]], add=True)
    plsc.subcore_barrier()

    pltpu.sync_copy(acc_sh.at[pl.ds(s * 40, 40)], degbuf_v)

    def expand(r, _):
        v16 = degbuf_v[r, :]
        base = r * 512
        for j in range(32):
            plsc.store_scatter(buf32_v, [iota16 * 32 + (base + j)], v16)
        return 0

    lax.fori_loop(0, 40, expand, 0)
    pltpu.sync_copy(buf32_v, out_hbm.at[c, pl.ds(s * 20480, 20480)])


def _sc_deg(dst3):
    mesh = plsc.VectorSubcoreMesh(core_axis_name="c", subcore_axis_name="s")
    return pl.kernel(
        _deg_kernel,
        mesh=mesh,
        compiler_params=pltpu.CompilerParams(use_tc_tiling_on_sc=False,
                                             needs_layout_passes=False),
        out_type=jax.ShapeDtypeStruct((NC, NP * 32), jnp.float32),
        scratch_types=[
            pltpu.VMEM((CHUNKS, 128), jnp.int32),
            pltpu.VMEM((640, 16), jnp.float32),
            pltpu.VMEM((20480,), jnp.float32),
            pltpu.VMEM((40, 16), jnp.float32),
            pltpu.VMEM((5, 128), jnp.int32),
            pltpu.VMEM((40, 16), jnp.float32),
            pltpu.VMEM_SHARED((640, 16), jnp.float32),
        ],
    )(dst3)


NR = NP // 4


def _kron4(wsub, rows):
    t = jnp.concatenate([wsub] * 4, axis=0)
    t = jnp.concatenate([t] * 4, axis=1)
    ri = lax.broadcasted_iota(jnp.int32, t.shape, 0) // (rows // 4)
    ci = lax.broadcasted_iota(jnp.int32, t.shape, 1) // 32
    return jnp.where(ri == ci, t, 0.0)


def _dinv_packed(degp_ref):
    deg = degp_ref[0] + degp_ref[1] + 1.0
    return lax.rsqrt(deg)


def _g0_body(x_ref, w_ref, degp_ref, o_ref):
    dinv = _dinv_packed(degp_ref)
    x2 = x_ref[...]
    for h in range(2):
        bd = _kron4(w_ref[:, h * DH:(h + 1) * DH], 512)
        o_ref[h] = dinv * jnp.dot(x2, bd, preferred_element_type=jnp.float32)


def _edges_body(ei_ref, src_ref, dst_ref):
    src_ref[pl.ds(0, E)] = ei_ref[0, :]
    dst_ref[pl.ds(0, E)] = ei_ref[1, :]
    src_ref[pl.ds(E, EP - E)] = jnp.full((EP - E,), N_NODES, jnp.int32)
    dst_ref[pl.ds(E, EP - E)] = jnp.full((EP - E,), NP - 1, jnp.int32)


def _tc_edges(edge_index):
    return pl.pallas_call(
        _edges_body,
        out_shape=[
            jax.ShapeDtypeStruct((EP,), jnp.int32),
            jax.ShapeDtypeStruct((EP,), jnp.int32),
        ],
    )(edge_index)


def _tc_g0(x2, W0, degp):
    return pl.pallas_call(
        _g0_body,
        out_shape=jax.ShapeDtypeStruct((NC, NR, 128), jnp.float32),
    )(x2, W0, degp)


def _relu_halves(s_ref, g_ref, dinv, b_ref):
    rs = []
    for h in range(2):
        bh = jnp.concatenate([b_ref[:, h * DH:(h + 1) * DH]] * 4, axis=1)
        rs.append(jnp.maximum(dinv * (s_ref[h] + g_ref[h]) + bh, 0.0))
    return rs


def _mid_body(s_ref, g_ref, degp_ref, b_ref, w_ref, o_ref):
    dinv = _dinv_packed(degp_ref)
    r = _relu_halves(s_ref, g_ref, dinv, b_ref)
    for h in range(2):
        acc = jnp.zeros((NR, 128), jnp.float32)
        for i in range(2):
            bd = _kron4(w_ref[i * DH:(i + 1) * DH, h * DH:(h + 1) * DH], 128)
            acc += jnp.dot(r[i], bd, preferred_element_type=jnp.float32)
        o_ref[h] = dinv * acc


def _tc_mid(S, g, degp, b, W):
    return pl.pallas_call(
        _mid_body,
        out_shape=jax.ShapeDtypeStruct((NC, NR, 128), jnp.float32),
    )(S, g, degp, b, W)


def _final_body(s_ref, g_ref, degp_ref, b_ref, batchp_ref, wlin_ref, blin_ref,
                o_ref):
    dinv = _dinv_packed(degp_ref)
    r = _relu_halves(s_ref, g_ref, dinv, b_ref)
    gid = lax.broadcasted_iota(jnp.int32, (NUM_GRAPHS, NR), 0)
    sums = []
    cnts = jnp.zeros((NUM_GRAPHS, 1), jnp.float32)
    for k in range(4):
        oh = (gid == batchp_ref[k:k + 1, :]).astype(jnp.float32)
        sums.append([jnp.dot(oh, r[h][:, k * DH:(k + 1) * DH],
                             preferred_element_type=jnp.float32)
                     for h in range(2)])
        cnts += jnp.sum(oh, axis=1, keepdims=True)
    pooled = jnp.concatenate(
        [sums[0][0] + sums[1][0] + sums[2][0] + sums[3][0],
         sums[0][1] + sums[1][1] + sums[2][1] + sums[3][1]],
        axis=1) / jnp.maximum(cnts, 1.0)
    o_ref[...] = jnp.dot(pooled, wlin_ref[...],
                         preferred_element_type=jnp.float32) + blin_ref[...]


def _tc_final(S, g, degp, b, batchp, Wlin, blin):
    return pl.pallas_call(
        _final_body,
        out_shape=jax.ShapeDtypeStruct((NUM_GRAPHS, Wlin.shape[1]),
                                       jnp.float32),
    )(S, g, degp, b, batchp, Wlin, blin)


@jax.jit
def kernel(x, edge_index, batch, W0, b0, W1, b1, W2, b2, Wlin, blin):
    n = x.shape[0]
    xp = jnp.zeros((NP, x.shape[1]), x.dtype).at[:n].set(x)
    src, dst = _tc_edges(edge_index)
    src3 = src.reshape(NS, CHUNKS, 128)
    dst3 = dst.reshape(NS, CHUNKS, 128)
    bb = jnp.full((NP,), NUM_GRAPHS + 7, jnp.int32).at[:n].set(batch)
    batchp = bb.reshape(NR, 4).T

    def to_sc(a):
        return a.reshape(NC, NP, DH)

    def to_tc(a):
        return a.reshape(NC, NR, 128)

    degp = _sc_deg(dst3).reshape(NC, NR, 128)
    g0 = _tc_g0(xp.reshape(NR, 512), W0, degp)
    S0 = to_tc(_sc_scatter(to_sc(g0), src3, dst3))
    g1 = _tc_mid(S0, g0, degp, b0.reshape(1, D), W1)
    S1 = to_tc(_sc_scatter(to_sc(g1), src3, dst3))
    g2 = _tc_mid(S1, g1, degp, b1.reshape(1, D), W2)
    S2 = to_tc(_sc_scatter(to_sc(g2), src3, dst3))
    out = _tc_final(S2, g2, degp, b2.reshape(1, D), batchp,
                    Wlin, blin.reshape(1, -1))
    return out
